# baseline (device time: 95063 ns/iter reference)
import os

import jax
import jax.numpy as jnp
from jax import lax
from jax.experimental import pallas as pl
from jax.experimental.pallas import tpu as pltpu

N_Y = 4
_NO_FOLD = bool(os.environ.get("KERNEL_NO_FOLD"))


def kernel(Q, K, V):
    b, s, h, d = Q.shape
    scale = d ** -0.5

    def body(q_ref, k_ref, v_ref, o_ref, kv_all, acc_buf, l_buf,
             out_send, out_recv, y_send, y_recv, x_send, x_recv):
        my_x = lax.axis_index("x")
        my_y = lax.axis_index("y")
        my_z = lax.axis_index("z")
        mb = my_x
        px = 1 - my_x

        for hh in range(h):
            kv_all[my_y, 0, :, d * hh:d * (hh + 1)] = (
                k_ref[mb, :, hh, :].astype(jnp.bfloat16))
            kv_all[my_y, 1, :, d * hh:d * (hh + 1)] = (
                v_ref[mb, :, hh, :].astype(jnp.bfloat16))

        s1 = jnp.where(my_y == 0, 1, my_y - 1)
        s2 = jnp.where(my_y == 0, 2, jnp.where(my_y == N_Y - 1, 1, my_y + 1))
        s3 = 6 - my_y - s1 - s2
        srcs = (s1, s2, s3)

        barrier = pltpu.get_barrier_semaphore()
        for sy in srcs:
            pl.semaphore_signal(barrier, inc=1, device_id=(my_x, sy, my_z),
                                device_id_type=pl.DeviceIdType.MESH)
        pl.semaphore_signal(barrier, inc=1, device_id=(px, my_y, my_z),
                            device_id_type=pl.DeviceIdType.MESH)
        pl.semaphore_wait(barrier, N_Y)

        y_descs = []
        for k, dy in enumerate(srcs):
            slot = my_y - (my_y > dy).astype(jnp.int32)
            desc = pltpu.make_async_remote_copy(
                src_ref=kv_all.at[my_y],
                dst_ref=kv_all.at[my_y],
                send_sem=y_send.at[k],
                recv_sem=y_recv.at[slot],
                device_id=(my_x, dy, my_z),
                device_id_type=pl.DeviceIdType.MESH,
            )
            desc.start()
            y_descs.append(desc)

        def fold(o, init, local=False):
            if _NO_FOLD:
                if init:
                    l_buf[:, :] = jnp.ones((s, h), jnp.float32)
                    acc_buf[:, :] = jnp.zeros((s, h * d), jnp.float32)
                return
            for hh in range(h):
                cols = slice(d * hh, d * (hh + 1))
                if local:
                    q_mat = q_ref[mb, :, hh, :]
                    k_sl = k_ref[mb, :, hh, :]
                    v_sl = v_ref[mb, :, hh, :]
                else:
                    q_mat = q_ref[mb, :, hh, :].astype(jnp.bfloat16)
                    k_sl = kv_all[o, 0, :, cols]
                    v_sl = kv_all[o, 1, :, cols]
                sc = lax.dot_general(
                    q_mat, k_sl,
                    (((1,), (1,)), ((), ())),
                    preferred_element_type=jnp.float32,
                ) * scale
                p = jnp.exp(sc)
                pv = lax.dot_general(
                    p if local else p.astype(jnp.bfloat16), v_sl,
                    (((1,), (0,)), ((), ())),
                    preferred_element_type=jnp.float32,
                )
                ps = jnp.sum(p, axis=-1, keepdims=True)
                if init:
                    l_buf[:, hh:hh + 1] = ps
                    acc_buf[:, cols] = pv
                else:
                    l_buf[:, hh:hh + 1] = l_buf[:, hh:hh + 1] + ps
                    acc_buf[:, cols] = acc_buf[:, cols] + pv

        fold(my_y, init=True, local=True)
        for sy in srcs:
            slot = sy - (sy > my_y).astype(jnp.int32)
            pltpu.make_async_remote_copy(
                src_ref=kv_all.at[sy],
                dst_ref=kv_all.at[sy],
                send_sem=y_send.at[0],
                recv_sem=y_recv.at[slot],
                device_id=(my_x, sy, my_z),
                device_id_type=pl.DeviceIdType.MESH,
            ).wait_recv()
            fold(sy, init=False)

        hp = h // 2
        xfers = []
        for half in range(2):
            for hh in range(half * hp, (half + 1) * hp):
                cols = slice(d * hh, d * (hh + 1))
                val = acc_buf[:, cols] / l_buf[:, hh:hh + 1]
                o_ref[mb, :, hh, :] = val
                out_send[:, cols] = val.astype(jnp.bfloat16)
            half_cols = slice(half * hp * d, (half + 1) * hp * d)
            xfer = pltpu.make_async_remote_copy(
                src_ref=out_send.at[:, half_cols],
                dst_ref=out_recv.at[:, half_cols],
                send_sem=x_send.at[half], recv_sem=x_recv.at[half],
                device_id=(px, my_y, my_z),
                device_id_type=pl.DeviceIdType.MESH,
            )
            xfer.start()
            xfers.append(xfer)

        for half, xfer in enumerate(xfers):
            xfer.wait_recv()
            for hh in range(half * hp, (half + 1) * hp):
                cols = slice(d * hh, d * (hh + 1))
                o_ref[1 - mb, :, hh, :] = out_recv[:, cols].astype(jnp.float32)
        for xfer in xfers:
            xfer.wait_send()

        for desc in y_descs:
            desc.wait_send()

    return pl.pallas_call(
        body,
        out_shape=jax.ShapeDtypeStruct((b, s, h, d), jnp.float32),
        in_specs=[
            pl.BlockSpec(memory_space=pltpu.VMEM),
            pl.BlockSpec(memory_space=pltpu.VMEM),
            pl.BlockSpec(memory_space=pltpu.VMEM),
        ],
        out_specs=pl.BlockSpec(memory_space=pltpu.VMEM),
        scratch_shapes=[
            pltpu.VMEM((N_Y, 2, s, h * d), jnp.bfloat16),
            pltpu.VMEM((s, h * d), jnp.float32),
            pltpu.VMEM((s, h), jnp.float32),
            pltpu.VMEM((s, h * d), jnp.bfloat16),
            pltpu.VMEM((s, h * d), jnp.bfloat16),
            pltpu.SemaphoreType.DMA((N_Y - 1,)),
            pltpu.SemaphoreType.DMA((N_Y - 1,)),
            pltpu.SemaphoreType.DMA((2,)),
            pltpu.SemaphoreType.DMA((2,)),
        ],
        compiler_params=pltpu.CompilerParams(
            collective_id=0,
            vmem_limit_bytes=60 * 1024 * 1024,
        ),
    )(Q, K, V)


# device time: 81087 ns/iter; 1.1724x vs baseline; 1.1724x over previous
import os

import jax
import jax.numpy as jnp
from jax import lax
from jax.experimental import pallas as pl
from jax.experimental.pallas import tpu as pltpu

N_Y = 4
_NO_FOLD = bool(os.environ.get("KERNEL_NO_FOLD"))


def kernel(Q, K, V):
    b, s, h, d = Q.shape
    scale = d ** -0.5

    def body(q_ref, k_ref, v_ref, o_ref, kv_all, acc_buf, l_buf,
             out_send, out_recv,
             e_send, e_recv, w_send, w_recv, x_send, x_recv):
        my_x = lax.axis_index("x")
        my_y = lax.axis_index("y")
        my_z = lax.axis_index("z")
        mb = my_x
        east = (my_y + 1) % N_Y
        west = (my_y - 1) % N_Y
        px = 1 - my_x
        has_e = my_y < N_Y - 1
        has_w = my_y > 0

        for hh in range(h):
            kv_all[my_y, 0, :, d * hh:d * (hh + 1)] = (
                k_ref[mb, :, hh, :].astype(jnp.bfloat16))
            kv_all[my_y, 1, :, d * hh:d * (hh + 1)] = (
                v_ref[mb, :, hh, :].astype(jnp.bfloat16))

        barrier = pltpu.get_barrier_semaphore()

        @pl.when(has_e)
        def _():
            pl.semaphore_signal(barrier, inc=1, device_id=(my_x, east, my_z),
                                device_id_type=pl.DeviceIdType.MESH)

        @pl.when(has_w)
        def _():
            pl.semaphore_signal(barrier, inc=1, device_id=(my_x, west, my_z),
                                device_id_type=pl.DeviceIdType.MESH)

        pl.semaphore_signal(barrier, inc=1, device_id=(px, my_y, my_z),
                            device_id_type=pl.DeviceIdType.MESH)
        n_nbrs = 1 + has_e.astype(jnp.int32) + has_w.astype(jnp.int32)
        pl.semaphore_wait(barrier, n_nbrs)

        def fold(o, init, local=False):
            if _NO_FOLD:
                if init:
                    l_buf[:, :] = jnp.ones((s, h), jnp.float32)
                    acc_buf[:, :] = jnp.zeros((s, h * d), jnp.float32)
                return
            for hh in range(h):
                cols = slice(d * hh, d * (hh + 1))
                q_mat = q_ref[mb, :, hh, :]
                if local:
                    k_sl = k_ref[mb, :, hh, :]
                    v_sl = v_ref[mb, :, hh, :]
                else:
                    k_sl = kv_all[o, 0, :, cols].astype(jnp.float32)
                    v_sl = kv_all[o, 1, :, cols].astype(jnp.float32)
                sc = lax.dot_general(
                    q_mat, k_sl,
                    (((1,), (1,)), ((), ())),
                    preferred_element_type=jnp.float32,
                ) * scale
                p = jnp.exp(sc)
                pv = lax.dot_general(
                    p, v_sl,
                    (((1,), (0,)), ((), ())),
                    preferred_element_type=jnp.float32,
                )
                ps = jnp.sum(p, axis=-1, keepdims=True)
                if init:
                    l_buf[:, hh:hh + 1] = ps
                    acc_buf[:, cols] = pv
                else:
                    l_buf[:, hh:hh + 1] = l_buf[:, hh:hh + 1] + ps
                    acc_buf[:, cols] = acc_buf[:, cols] + pv

        def rdma(src_o, dev_y, ss, rs):
            return pltpu.make_async_remote_copy(
                src_ref=kv_all.at[src_o % N_Y],
                dst_ref=kv_all.at[src_o % N_Y],
                send_sem=ss, recv_sem=rs,
                device_id=(my_x, dev_y % N_Y, my_z),
                device_id_type=pl.DeviceIdType.MESH,
            )

        def guarded(cond, fn):
            @pl.when(cond)
            def _():
                fn()

        sends = []

        for t in range(1, 5):
            if t <= 3:
                c_e = jnp.logical_and(has_e, t <= my_y + 1)
                d_e = rdma(my_y - t + 1, my_y + 1,
                           e_send.at[t - 1], e_recv.at[t - 1])
                guarded(c_e, d_e.start)
                sends.append((c_e, d_e))

                c_w = jnp.logical_and(has_w, my_y + t - 1 <= N_Y - 1)
                d_w = rdma(my_y + t - 1, my_y - 1,
                           w_send.at[t - 1], w_recv.at[t - 1])
                guarded(c_w, d_w.start)
                sends.append((c_w, d_w))

            if t == 1:
                fold(my_y, init=True, local=True)
            else:
                guarded(t - 1 <= my_y,
                        lambda t=t: fold((my_y - t + 1) % N_Y, False))
                guarded(t - 1 <= N_Y - 1 - my_y,
                        lambda t=t: fold((my_y + t - 1) % N_Y, False))

            if t <= 3:
                guarded(t <= my_y,
                        lambda t=t: rdma(my_y - t, 0,
                                         e_send.at[t - 1],
                                         e_recv.at[t - 1]).wait_recv())
                guarded(t <= N_Y - 1 - my_y,
                        lambda t=t: rdma(my_y + t, 0,
                                         w_send.at[t - 1],
                                         w_recv.at[t - 1]).wait_recv())

        hp = h // 2
        xfers = []
        for half in range(2):
            for hh in range(half * hp, (half + 1) * hp):
                cols = slice(d * hh, d * (hh + 1))
                val = acc_buf[:, cols] / l_buf[:, hh:hh + 1]
                o_ref[mb, :, hh, :] = val
                out_send[:, cols] = val.astype(jnp.bfloat16)
            half_cols = slice(half * hp * d, (half + 1) * hp * d)
            xfer = pltpu.make_async_remote_copy(
                src_ref=out_send.at[:, half_cols],
                dst_ref=out_recv.at[:, half_cols],
                send_sem=x_send.at[half], recv_sem=x_recv.at[half],
                device_id=(px, my_y, my_z),
                device_id_type=pl.DeviceIdType.MESH,
            )
            xfer.start()
            xfers.append(xfer)

        for half, xfer in enumerate(xfers):
            xfer.wait_recv()
            for hh in range(half * hp, (half + 1) * hp):
                cols = slice(d * hh, d * (hh + 1))
                o_ref[1 - mb, :, hh, :] = out_recv[:, cols].astype(jnp.float32)
        for xfer in xfers:
            xfer.wait_send()

        for cond, desc in sends:
            guarded(cond, desc.wait_send)

    return pl.pallas_call(
        body,
        out_shape=jax.ShapeDtypeStruct((b, s, h, d), jnp.float32),
        in_specs=[
            pl.BlockSpec(memory_space=pltpu.VMEM),
            pl.BlockSpec(memory_space=pltpu.VMEM),
            pl.BlockSpec(memory_space=pltpu.VMEM),
        ],
        out_specs=pl.BlockSpec(memory_space=pltpu.VMEM),
        scratch_shapes=[
            pltpu.VMEM((N_Y, 2, s, h * d), jnp.bfloat16),
            pltpu.VMEM((s, h * d), jnp.float32),
            pltpu.VMEM((s, h), jnp.float32),
            pltpu.VMEM((s, h * d), jnp.bfloat16),
            pltpu.VMEM((s, h * d), jnp.bfloat16),
            pltpu.SemaphoreType.DMA((N_Y - 1,)),
            pltpu.SemaphoreType.DMA((N_Y - 1,)),
            pltpu.SemaphoreType.DMA((N_Y - 1,)),
            pltpu.SemaphoreType.DMA((N_Y - 1,)),
            pltpu.SemaphoreType.DMA((2,)),
            pltpu.SemaphoreType.DMA((2,)),
        ],
        compiler_params=pltpu.CompilerParams(
            collective_id=0,
            vmem_limit_bytes=60 * 1024 * 1024,
        ),
    )(Q, K, V)
